# baseline (speedup 1.0000x reference)
import functools

import jax
import jax.numpy as jnp
from jax import lax
from jax.experimental import pallas as pl
from jax.experimental.pallas import tpu as pltpu
from jax.experimental.pallas import tpu_sc as plsc

NC, NS, L = 2, 16, 16
NW = NC * NS

EPS = 1e-6


def _make_score_kernel(T, N, D, E):
    TOT = T * 2 * E
    PER_W = TOT // NW
    C = 80
    NCHUNK = PER_W // C
    assert PER_W * NW == TOT and NCHUNK * C == PER_W and D == 16 * L

    mesh = plsc.VectorSubcoreMesh(core_axis_name="c", subcore_axis_name="s")

    @functools.partial(
        pl.kernel,
        mesh=mesh,
        out_type=jax.ShapeDtypeStruct((TOT, L), jnp.float32),
        compiler_params=pltpu.CompilerParams(use_tc_tiling_on_sc=False,
                                             needs_layout_passes=False),
        scratch_types=[
            [pltpu.VMEM((C,), jnp.int32) for _ in range(4)],
            [pltpu.VMEM((C,), jnp.int32) for _ in range(4)],
            [pltpu.VMEM((C, D), jnp.bfloat16) for _ in range(4)],
            [pltpu.VMEM((C, D), jnp.bfloat16) for _ in range(4)],
            pltpu.VMEM((C, L), jnp.float32),
            [pltpu.SemaphoreType.DMA for _ in range(4)],
            [pltpu.SemaphoreType.DMA for _ in range(4)],
            [pltpu.SemaphoreType.DMA for _ in range(4)],
            [pltpu.SemaphoreType.DMA for _ in range(4)],
        ],
    )
    def score_kernel(z_hbm, src_hbm, dst_hbm, out_hbm,
                     SI, DI, SR, DR, scv, semsi, semdi, semsr, semdr):
        wid = lax.axis_index("s") * NC + lax.axis_index("c")
        base = wid * PER_W

        def idx_start(ci, k):
            off = base + ci * C
            pltpu.async_copy(src_hbm.at[pl.ds(off, C)], SI[k], semsi[k])
            pltpu.async_copy(dst_hbm.at[pl.ds(off, C)], DI[k], semdi[k])

        def idx_wait(ci, k):
            off = base + ci * C
            pltpu.make_async_copy(src_hbm.at[pl.ds(off, C)], SI[k],
                                  semsi[k]).wait()
            pltpu.make_async_copy(dst_hbm.at[pl.ds(off, C)], DI[k],
                                  semdi[k]).wait()

        def row_start(ci, k):
            pltpu.async_copy(z_hbm.at[SI[k]], SR[k], semsr[k])
            pltpu.async_copy(z_hbm.at[DI[k]], DR[k], semdr[k])

        def row_wait(ci, k):
            pltpu.make_async_copy(z_hbm.at[SI[k]], SR[k], semsr[k]).wait()
            pltpu.make_async_copy(z_hbm.at[DI[k]], DR[k], semdr[k]).wait()

        def compute(ci, k):
            srows, drows = SR[k], DR[k]
            off = base + ci * C

            def edge(e, carry2):
                parts = []
                for m in range(4):
                    pr = (srows[e, pl.ds(m * 64, 2 * L)]
                          * drows[e, pl.ds(m * 64, 2 * L)])
                    pa, pb = plsc.unpack(pr,
                                         format=plsc.PackFormat.INTERLEAVED)
                    p0 = pa + pb
                    pr = (srows[e, pl.ds(m * 64 + 2 * L, 2 * L)]
                          * drows[e, pl.ds(m * 64 + 2 * L, 2 * L)])
                    pa, pb = plsc.unpack(pr,
                                         format=plsc.PackFormat.INTERLEAVED)
                    parts.append(p0 + (pa + pb))
                scv[e] = (parts[0] + parts[1]) + (parts[2] + parts[3])
                return carry2

            lax.fori_loop(0, C, edge, 0, unroll=4)
            pltpu.sync_copy(scv, out_hbm.at[pl.ds(off, C)])

        for k in range(3):
            idx_start(k, k)
        for k in range(3):
            idx_wait(k, k)
            row_start(k, k)
        idx_start(3, 3)

        def body(q, carry):
            c0 = 4 * q
            for ph in range(4):
                c = c0 + ph
                row_wait(c, ph)

                @pl.when(c + 4 < NCHUNK)
                def _():
                    idx_start(c + 4, ph)

                @pl.when(c + 3 < NCHUNK)
                def _():
                    idx_wait(c + 3, (ph + 3) % 4)
                    row_start(c + 3, (ph + 3) % 4)

                compute(c, ph)
            return carry

        lax.fori_loop(0, NCHUNK // 4, body, 0)

    return score_kernel


def _loss_body(T, E, nblk, x_ref, o_ref):
    si = pl.program_id(0)
    ci = pl.program_id(1)

    @pl.when((si == 0) & (ci == 0))
    def _():
        o_ref[0, 0] = 0.0

    x = x_ref[...]
    sc = jnp.sum(x[0], axis=1)
    sig = 1.0 / (1.0 + jnp.exp(-sc))
    pos_t = jnp.log(sig + EPS)
    neg_t = jnp.log(1.0 - sig + EPS)
    val = jnp.sum(jnp.where(si % 2 == 0, pos_t, neg_t))
    o_ref[0, 0] += val

    @pl.when((si == 2 * T - 1) & (ci == nblk - 1))
    def _():
        o_ref[0, 0] = o_ref[0, 0] * (-1.0 / (T * E))


def kernel(ps, ns, zs):
    T, N, D = zs.shape
    E = ps.shape[2]
    L16 = 16

    zf = zs.astype(jnp.bfloat16).reshape(T * N, D)
    offs = (jnp.arange(T, dtype=jnp.int32) * N)[:, None, None]
    src = (jnp.stack([ps[:, 0, :], ns[:, 0, :]], axis=1).astype(jnp.int32)
           + offs).reshape(-1)
    dst = (jnp.stack([ps[:, 1, :], ns[:, 1, :]], axis=1).astype(jnp.int32)
           + offs).reshape(-1)

    partials = _make_score_kernel(T, N, D, E)(zf, src, dst)
    x = partials.reshape(T * 2, E, L16)

    BE = 2000
    nblk = E // BE
    loss = pl.pallas_call(
        functools.partial(_loss_body, T, E, nblk),
        grid=(T * 2, nblk),
        out_shape=jax.ShapeDtypeStruct((1, 1), jnp.float32),
        in_specs=[
            pl.BlockSpec((1, BE, L16), lambda s, c: (s, c, 0)),
        ],
        out_specs=pl.BlockSpec(memory_space=pltpu.SMEM,
                               index_map=lambda s, c: (0, 0)),
    )(x)
    return loss.reshape(1)


# speedup vs baseline: 3.2492x; 3.2492x over previous
import functools

import jax
import jax.numpy as jnp
from jax import lax
from jax.experimental import pallas as pl
from jax.experimental.pallas import tpu as pltpu
from jax.experimental.pallas import tpu_sc as plsc

NC, NS, L = 2, 16, 16
NW = NC * NS

EPS = 1e-6


def _make_score_kernel(T, N, D, E):
    TOT = T * 2 * E
    PER_W = TOT // NW
    C = 80
    NCHUNK = PER_W // C
    assert PER_W * NW == TOT and NCHUNK * C == PER_W and D == 16 * L

    mesh = plsc.VectorSubcoreMesh(core_axis_name="c", subcore_axis_name="s")

    @functools.partial(
        pl.kernel,
        mesh=mesh,
        out_type=jax.ShapeDtypeStruct((TOT,), jnp.float32),
        compiler_params=pltpu.CompilerParams(use_tc_tiling_on_sc=False,
                                             needs_layout_passes=False),
        scratch_types=[
            [pltpu.VMEM((C,), jnp.int32) for _ in range(4)],
            [pltpu.VMEM((C,), jnp.int32) for _ in range(4)],
            [pltpu.VMEM((C, D), jnp.bfloat16) for _ in range(4)],
            [pltpu.VMEM((C, D), jnp.bfloat16) for _ in range(4)],
            pltpu.VMEM((C,), jnp.float32),
            [pltpu.SemaphoreType.DMA for _ in range(4)],
            [pltpu.SemaphoreType.DMA for _ in range(4)],
            [pltpu.SemaphoreType.DMA for _ in range(4)],
            [pltpu.SemaphoreType.DMA for _ in range(4)],
        ],
    )
    def score_kernel(z_hbm, src_hbm, dst_hbm, out_hbm,
                     SI, DI, SR, DR, scv, semsi, semdi, semsr, semdr):
        wid = lax.axis_index("s") * NC + lax.axis_index("c")
        base = wid * PER_W

        def idx_start(ci, k):
            off = base + ci * C
            pltpu.async_copy(src_hbm.at[pl.ds(off, C)], SI[k], semsi[k])
            pltpu.async_copy(dst_hbm.at[pl.ds(off, C)], DI[k], semdi[k])

        def idx_wait(ci, k):
            off = base + ci * C
            pltpu.make_async_copy(src_hbm.at[pl.ds(off, C)], SI[k],
                                  semsi[k]).wait()
            pltpu.make_async_copy(dst_hbm.at[pl.ds(off, C)], DI[k],
                                  semdi[k]).wait()

        def row_start(ci, k):
            pltpu.async_copy(z_hbm.at[SI[k]], SR[k], semsr[k])
            pltpu.async_copy(z_hbm.at[DI[k]], DR[k], semdr[k])

        def row_wait(ci, k):
            pltpu.make_async_copy(z_hbm.at[SI[k]], SR[k], semsr[k]).wait()
            pltpu.make_async_copy(z_hbm.at[DI[k]], DR[k], semdr[k]).wait()

        ii = lax.iota(jnp.int32, L)

        def compute(ci, k):
            srows, drows = SR[k], DR[k]
            off = base + ci * C

            def group(g, carry2):
                base_e = g * L

                def edge(j, svec):
                    e = base_e + j
                    parts = []
                    for m in range(4):
                        pr = (srows[e, pl.ds(m * 64, 2 * L)]
                              * drows[e, pl.ds(m * 64, 2 * L)])
                        pa, pb = plsc.unpack(pr,
                                             format=plsc.PackFormat.INTERLEAVED)
                        p0 = pa + pb
                        pr = (srows[e, pl.ds(m * 64 + 2 * L, 2 * L)]
                              * drows[e, pl.ds(m * 64 + 2 * L, 2 * L)])
                        pa, pb = plsc.unpack(pr,
                                             format=plsc.PackFormat.INTERLEAVED)
                        parts.append(p0 + (pa + pb))
                    acc = (parts[0] + parts[1]) + (parts[2] + parts[3])
                    return jnp.where(ii == j, jnp.sum(acc), svec)

                svec = lax.fori_loop(0, L, edge, jnp.zeros((L,), jnp.float32),
                                     unroll=4)
                scv[pl.ds(base_e, L)] = svec
                return carry2

            lax.fori_loop(0, C // L, group, 0)
            pltpu.sync_copy(scv, out_hbm.at[pl.ds(off, C)])

        for k in range(3):
            idx_start(k, k)
        for k in range(3):
            idx_wait(k, k)
            row_start(k, k)
        idx_start(3, 3)

        def body(q, carry):
            c0 = 4 * q
            for ph in range(4):
                c = c0 + ph
                row_wait(c, ph)

                @pl.when(c + 4 < NCHUNK)
                def _():
                    idx_start(c + 4, ph)

                @pl.when(c + 3 < NCHUNK)
                def _():
                    idx_wait(c + 3, (ph + 3) % 4)
                    row_start(c + 3, (ph + 3) % 4)

                compute(c, ph)
            return carry

        lax.fori_loop(0, NCHUNK // 4, body, 0)

    return score_kernel


def _loss_body(T, E, p_ref, n_ref, o_ref):
    p = p_ref[...]
    n = n_ref[...]
    sp = 1.0 / (1.0 + jnp.exp(-p))
    sn = 1.0 / (1.0 + jnp.exp(-n))
    tp = jnp.log(sp + EPS)
    tn = jnp.log(1.0 - sn + EPS)
    o_ref[0, 0] = -(jnp.sum(tp) + jnp.sum(tn)) / (T * E)


def kernel(ps, ns, zs):
    T, N, D = zs.shape
    E = ps.shape[2]

    zf = zs.astype(jnp.bfloat16).reshape(T * N, D)
    offs = (jnp.arange(T, dtype=jnp.int32) * N)[:, None, None]
    src = (jnp.stack([ps[:, 0, :], ns[:, 0, :]], axis=1).astype(jnp.int32)
           + offs).reshape(-1)
    dst = (jnp.stack([ps[:, 1, :], ns[:, 1, :]], axis=1).astype(jnp.int32)
           + offs).reshape(-1)

    scores = _make_score_kernel(T, N, D, E)(zf, src, dst)
    sc4 = scores.reshape(T, 2, E)
    pos = sc4[:, 0, :]
    neg = sc4[:, 1, :]

    loss = pl.pallas_call(
        functools.partial(_loss_body, T, E),
        out_shape=jax.ShapeDtypeStruct((1, 1), jnp.float32),
        in_specs=[
            pl.BlockSpec(memory_space=pltpu.VMEM),
            pl.BlockSpec(memory_space=pltpu.VMEM),
        ],
        out_specs=pl.BlockSpec(memory_space=pltpu.SMEM),
    )(pos, neg)
    return loss.reshape(1)
